# baseline (speedup 1.0000x reference)
import functools

import jax
import jax.numpy as jnp
from jax import lax
from jax.experimental import pallas as pl
from jax.experimental.pallas import tpu as pltpu
from jax.experimental.pallas import tpu_sc as plsc

N_NODES = 10000
N_EDGES = 320000
D = 128
NW = 32
NODES_PER_W = 320
NP = NW * NODES_PER_W
EP = N_EDGES + 2560

EDGE_BM = 2560
NODE_BM = 2048

TSTRIDE = NODES_PER_W + 8
TROWS = 16 * TSTRIDE


_DN = (((1,), (1,)), ((), ()))


def _mm2_body(x_ref, wa_ref, ba_ref, wb_ref, bb_ref, oa_ref, ob_ref):
    xv = x_ref[...]
    oa_ref[...] = lax.dot_general(xv, wa_ref[...], _DN,
                                  preferred_element_type=jnp.float32) + ba_ref[...]
    ob_ref[...] = lax.dot_general(xv, wb_ref[...], _DN,
                                  preferred_element_type=jnp.float32) + bb_ref[...]


def _node_mm2(x, wa, ba, wb, bb):
    grid = (x.shape[0] // NODE_BM,)
    return pl.pallas_call(
        _mm2_body,
        grid=grid,
        in_specs=[
            pl.BlockSpec((NODE_BM, D), lambda i: (i, 0)),
            pl.BlockSpec((D, D), lambda i: (0, 0)),
            pl.BlockSpec((1, D), lambda i: (0, 0)),
            pl.BlockSpec((D, D), lambda i: (0, 0)),
            pl.BlockSpec((1, D), lambda i: (0, 0)),
        ],
        out_specs=[
            pl.BlockSpec((NODE_BM, D), lambda i: (i, 0)),
            pl.BlockSpec((NODE_BM, D), lambda i: (i, 0)),
        ],
        out_shape=[
            jax.ShapeDtypeStruct((x.shape[0], D), jnp.float32),
            jax.ShapeDtypeStruct((x.shape[0], D), jnp.float32),
        ],
    )(x, wa, ba.reshape(1, D), wb, bb.reshape(1, D))


def _mm1_body(x_ref, w_ref, b_ref, o_ref):
    o_ref[...] = lax.dot_general(x_ref[...], w_ref[...], _DN,
                                 preferred_element_type=jnp.float32) + b_ref[...]


def _node_mm1(x, w, b):
    grid = (x.shape[0] // NODE_BM,)
    return pl.pallas_call(
        _mm1_body,
        grid=grid,
        in_specs=[
            pl.BlockSpec((NODE_BM, D), lambda i: (i, 0)),
            pl.BlockSpec((D, D), lambda i: (0, 0)),
            pl.BlockSpec((1, D), lambda i: (0, 0)),
        ],
        out_specs=pl.BlockSpec((NODE_BM, D), lambda i: (i, 0)),
        out_shape=jax.ShapeDtypeStruct((x.shape[0], D), jnp.float32),
    )(x, w, b.reshape(1, D))


def _ea_body(ea8_ref, wbd_ref, o_ref):
    t = lax.dot_general(ea8_ref[...], wbd_ref[...], (((1,), (0,)), ((), ())),
                        preferred_element_type=jnp.float32)
    o_ref[...] = t.reshape(EDGE_BM, D)


def _ea_kernel(ea8, wbd):
    grid = (N_EDGES // EDGE_BM,)
    return pl.pallas_call(
        _ea_body,
        grid=grid,
        in_specs=[
            pl.BlockSpec((EDGE_BM // 8, D), lambda i: (i, 0)),
            pl.BlockSpec((D, 8 * D), lambda i: (0, 0)),
        ],
        out_specs=pl.BlockSpec((EDGE_BM, D), lambda i: (i, 0)),
        out_shape=jax.ShapeDtypeStruct((N_EDGES, D), jnp.float32),
    )(ea8, wbd)


def _relu_mm2_body(h0_ref, w_ref, oh_ref, omm_ref):
    t = jnp.maximum(h0_ref[...], 0.0)
    oh_ref[...] = t
    omm_ref[...] = lax.dot_general(t, w_ref[...], _DN,
                                   preferred_element_type=jnp.float32)


def _relu_mm2(h0, w):
    grid = (EP // EDGE_BM,)
    return pl.pallas_call(
        _relu_mm2_body,
        grid=grid,
        in_specs=[
            pl.BlockSpec((EDGE_BM, D), lambda i: (i, 0)),
            pl.BlockSpec((D, D), lambda i: (0, 0)),
        ],
        out_specs=[
            pl.BlockSpec((EDGE_BM, D), lambda i: (i, 0)),
            pl.BlockSpec((EDGE_BM, D), lambda i: (i, 0)),
        ],
        out_shape=[
            jax.ShapeDtypeStruct((EP, D), jnp.float32),
            jax.ShapeDtypeStruct((EP, D), jnp.float32),
        ],
    )(h0, w)


def _ewmm_body(h0_ref, g1_ref, g2_ref, m_ref, c_ref, w_ref, oh_ref, omm_ref):
    g2e = c_ref[...] + m_ref[...] * (g2_ref[...] - c_ref[...])
    t = jnp.maximum(h0_ref[...] + g1_ref[...] - g2e, 0.0)
    oh_ref[...] = t
    omm_ref[...] = lax.dot_general(t, w_ref[...], _DN,
                                   preferred_element_type=jnp.float32)


def _ewmm(h0, g1, g2, m, c, w):
    grid = (EP // EDGE_BM,)
    return pl.pallas_call(
        _ewmm_body,
        grid=grid,
        in_specs=[
            pl.BlockSpec((EDGE_BM, D), lambda i: (i, 0)),
            pl.BlockSpec((EDGE_BM, D), lambda i: (i, 0)),
            pl.BlockSpec((EDGE_BM, D), lambda i: (i, 0)),
            pl.BlockSpec((EDGE_BM, 1), lambda i: (i, 0)),
            pl.BlockSpec((1, D), lambda i: (0, 0)),
            pl.BlockSpec((D, D), lambda i: (0, 0)),
        ],
        out_specs=[
            pl.BlockSpec((EDGE_BM, D), lambda i: (i, 0)),
            pl.BlockSpec((EDGE_BM, D), lambda i: (i, 0)),
        ],
        out_shape=[
            jax.ShapeDtypeStruct((EP, D), jnp.float32),
            jax.ShapeDtypeStruct((EP, D), jnp.float32),
        ],
    )(h0, g1, g2, m, c, w)


def _ew_body(h0_ref, g1_ref, g2_ref, m_ref, c_ref, o_ref):
    g2e = c_ref[...] + m_ref[...] * (g2_ref[...] - c_ref[...])
    o_ref[...] = jnp.maximum(h0_ref[...] + g1_ref[...] - g2e, 0.0)


def _ew(h0, g1, g2, m, c):
    grid = (EP // EDGE_BM,)
    return pl.pallas_call(
        _ew_body,
        grid=grid,
        in_specs=[
            pl.BlockSpec((EDGE_BM, D), lambda i: (i, 0)),
            pl.BlockSpec((EDGE_BM, D), lambda i: (i, 0)),
            pl.BlockSpec((EDGE_BM, D), lambda i: (i, 0)),
            pl.BlockSpec((EDGE_BM, 1), lambda i: (i, 0)),
            pl.BlockSpec((1, D), lambda i: (0, 0)),
        ],
        out_specs=pl.BlockSpec((EDGE_BM, D), lambda i: (i, 0)),
        out_shape=jax.ShapeDtypeStruct((EP, D), jnp.float32),
    )(h0, g1, g2, m, c)


def _head_body(x_ref, m_ref, tx_ref, wox_ref, wom_ref, bo_ref,
               c1_ref, c1b_ref, c2_ref, c2b_ref, c3_ref, c3b_ref,
               r1_ref, r1b_ref, r2_ref, r2b_ref, r3_ref, r3b_ref, o_ref):
    m = m_ref[...]
    msum = jnp.sum(m, axis=1, keepdims=True)
    muse = jnp.where(msum == 0.0, tx_ref[...], m)
    emb = jnp.maximum(
        lax.dot_general(x_ref[...], wox_ref[...], _DN, preferred_element_type=jnp.float32)
        + lax.dot_general(muse, wom_ref[...], _DN, preferred_element_type=jnp.float32)
        + bo_ref[...], 0.0)
    h1 = jnp.maximum(lax.dot_general(emb, c1_ref[...], _DN,
                                     preferred_element_type=jnp.float32) + c1b_ref[...], 0.0)
    h2 = jnp.maximum(lax.dot_general(h1, c2_ref[...], _DN,
                                     preferred_element_type=jnp.float32) + c2b_ref[...], 0.0)
    cls = lax.dot_general(h2, c3_ref[...], _DN,
                          preferred_element_type=jnp.float32) + c3b_ref[...]
    q1 = jnp.maximum(lax.dot_general(emb, r1_ref[...], _DN,
                                     preferred_element_type=jnp.float32) + r1b_ref[...], 0.0)
    q2 = jnp.maximum(lax.dot_general(q1, r2_ref[...], _DN,
                                     preferred_element_type=jnp.float32) + r2b_ref[...], 0.0)
    reg = lax.dot_general(q2, r3_ref[...], _DN,
                          preferred_element_type=jnp.float32) + r3b_ref[...]
    reg = jnp.clip(jax.nn.sigmoid(reg), 0.0, 14.0)
    col = lax.broadcasted_iota(jnp.int32, cls.shape, 1)
    o_ref[...] = jnp.where(col < 2, cls, jnp.where(col == 2, reg, 0.0))


def _head(x, m, tx, wox, wom, bo, c1, c1b, c2, c2b, c3p, c3bp,
          r1, r1b, r2, r2b, r3p, r3bp):
    grid = (NP // NODE_BM,)
    bs = lambda shp: pl.BlockSpec(shp, lambda i: (0, 0))
    return pl.pallas_call(
        _head_body,
        grid=grid,
        in_specs=[
            pl.BlockSpec((NODE_BM, D), lambda i: (i, 0)),
            pl.BlockSpec((NODE_BM, D), lambda i: (i, 0)),
            pl.BlockSpec((NODE_BM, D), lambda i: (i, 0)),
            bs((D, D)), bs((D, D)), bs((1, D)),
            bs((256, D)), bs((1, 256)), bs((64, 256)), bs((1, 64)),
            bs((D, 64)), bs((1, D)),
            bs((256, D)), bs((1, 256)), bs((64, 256)), bs((1, 64)),
            bs((D, 64)), bs((1, D)),
        ],
        out_specs=pl.BlockSpec((NODE_BM, D), lambda i: (i, 0)),
        out_shape=jax.ShapeDtypeStruct((NP, D), jnp.float32),
    )(x, m, tx, wox, wom, bo.reshape(1, D),
      c1, c1b.reshape(1, 256), c2, c2b.reshape(1, 64), c3p, c3bp.reshape(1, D),
      r1, r1b.reshape(1, 256), r2, r2b.reshape(1, 64), r3p, r3bp.reshape(1, D))


_MESH = functools.partial(plsc.VectorSubcoreMesh,
                          core_axis_name="c", subcore_axis_name="s")

_GCH = 400
_GSUB = ((0, 128), (128, 128), (256, 128), (384, 16))


def _sc_gather(table, idx, out_rows, name):
    d = table.shape[1]
    e = idx.shape[0]
    per_w = e // NW
    n_iter = per_w // _GCH

    def body(table_ref, idx_ref, out_ref, i0, i1, i2, i3, rowv, sem):
        c = lax.axis_index("c")
        s = lax.axis_index("s")
        wid = s * 2 + c
        base0 = wid * per_w
        ibufs = (i0, i1, i2, i3)

        def it(j, carry):
            off = base0 + j * _GCH
            for (k, ln), ib in zip(_GSUB, ibufs):
                pltpu.sync_copy(idx_ref.at[pl.ds(off + k, ln)], ib)
            cps = [pltpu.async_copy(table_ref.at[ib], rowv.at[pl.ds(k, ln)], sem)
                   for (k, ln), ib in zip(_GSUB, ibufs)]
            for cp in cps:
                cp.wait()
            pltpu.sync_copy(rowv, out_ref.at[pl.ds(off, _GCH)])
            return carry

        lax.fori_loop(0, n_iter, it, 0)

    return pl.kernel(
        body,
        out_type=jax.ShapeDtypeStruct((out_rows, d), jnp.float32),
        mesh=_MESH(),
        name=name,
        scratch_types=[
            pltpu.VMEM((128,), jnp.int32),
            pltpu.VMEM((128,), jnp.int32),
            pltpu.VMEM((128,), jnp.int32),
            pltpu.VMEM((16,), jnp.int32),
            pltpu.VMEM((_GCH, d), jnp.float32),
            pltpu.SemaphoreType.DMA,
        ],
    )(table, idx)


def _sc_gather_add(ta, ia, tb, ib_idx, out_rows):
    name = 'scgadd'
    d = ta.shape[1]
    e = ia.shape[0]
    per_w = e // NW
    n_iter = per_w // _GCH

    def body(ta_ref, ia_ref, tb_ref, ib_ref, out_ref, i0, i1, i2, i3, rowv, sem):
        c = lax.axis_index("c")
        s = lax.axis_index("s")
        wid = s * 2 + c
        base0 = wid * per_w
        ibufs = (i0, i1, i2, i3)

        def it(j, carry):
            off = base0 + j * _GCH
            for (k, ln), ib in zip(_GSUB, ibufs):
                pltpu.sync_copy(ia_ref.at[pl.ds(off + k, ln)], ib)
            cps = [pltpu.async_copy(ta_ref.at[ib], rowv.at[pl.ds(k, ln)], sem)
                   for (k, ln), ib in zip(_GSUB, ibufs)]
            for cp in cps:
                cp.wait()
            for (k, ln), ib in zip(_GSUB, ibufs):
                pltpu.sync_copy(ib_ref.at[pl.ds(off + k, ln)], ib)
            cps = [pltpu.async_copy(tb_ref.at[ib], rowv.at[pl.ds(k, ln)], sem,
                                    add=True)
                   for (k, ln), ib in zip(_GSUB, ibufs)]
            for cp in cps:
                cp.wait()
            pltpu.sync_copy(rowv, out_ref.at[pl.ds(off, _GCH)])
            return carry

        lax.fori_loop(0, n_iter, it, 0)

    return pl.kernel(
        body,
        out_type=jax.ShapeDtypeStruct((out_rows, d), jnp.float32),
        mesh=_MESH(),
        name=name,
        scratch_types=[
            pltpu.VMEM((128,), jnp.int32),
            pltpu.VMEM((128,), jnp.int32),
            pltpu.VMEM((128,), jnp.int32),
            pltpu.VMEM((16,), jnp.int32),
            pltpu.VMEM((_GCH, d), jnp.float32),
            pltpu.SemaphoreType.DMA,
        ],
    )(ta, ia, tb, ib_idx)


def _ext48(v0, v1, v2, i):
    z = jnp.int32(0)
    io = lax.iota(jnp.int32, 16)
    r0 = jnp.sum(jnp.where(io == i, v0, z))
    r1 = jnp.sum(jnp.where(io == (i - 16), v1, z))
    r2 = jnp.sum(jnp.where(io == (i - 32), v2, z))
    return r0 + r1 + r2


def _sc_segsum(rows, dstp_pad, bounds_pad, ztab, name):
    def body(rows_ref, dst_ref, bnd_ref, z_ref, out_ref,
             bndv, dstv, idxv, rowv, table):
        c = lax.axis_index("c")
        s = lax.axis_index("s")
        wid = s * 2 + c
        tb = TSTRIDE * s
        dump = tb + NODES_PER_W

        pltpu.sync_copy(z_ref.at[pl.ds(tb, TSTRIDE)], table.at[pl.ds(tb, TSTRIDE)])
        pltpu.sync_copy(bnd_ref.at[wid], bndv)
        bv = bndv[...]
        lo = bv[0]
        hi = bv[1]
        a = (lo // 128) * 128
        nch = (hi - a + 127) // 128
        shift = tb - NODES_PER_W * wid

        def it(j, carry):
            base = a + j * 128
            pltpu.sync_copy(rows_ref.at[pl.ds(base, 128), :], rowv)
            pltpu.sync_copy(dst_ref.at[pl.ds(base, 128)], dstv)
            for g in range(8):
                ids = base + 16 * g + lax.iota(jnp.int32, 16)
                dv = dstv[pl.ds(16 * g, 16)]
                ok = (ids >= lo) & (ids < hi)
                idxv[pl.ds(16 * g, 16)] = jnp.where(ok, dv + shift, dump)
            pltpu.sync_copy(rowv, table.at[idxv], add=True)
            return carry

        lax.fori_loop(0, nch, it, 0)
        pltpu.sync_copy(table.at[pl.ds(tb, NODES_PER_W)],
                        out_ref.at[pl.ds(wid * NODES_PER_W, NODES_PER_W)])

    return pl.kernel(
        body,
        out_type=jax.ShapeDtypeStruct((NP, D), jnp.float32),
        mesh=_MESH(),
        name=name,
        scratch_types=[
            pltpu.VMEM((16,), jnp.int32),
            pltpu.VMEM((128,), jnp.int32),
            pltpu.VMEM((128,), jnp.int32),
            pltpu.VMEM((128, D), jnp.float32),
            pltpu.VMEM_SHARED((TROWS, D), jnp.float32),
        ],
    )(rows, dstp_pad, bounds_pad, ztab)


def kernel(x, edge_index, edge_attr, dissociable_masks, pka_values,
           Wi_w, Wi_b, Wh_w, Wh_b, Wo_w, Wo_b, Wt_w, Wt_b,
           c1_w, c1_b, c2_w, c2_b, c3_w, c3_b,
           r1_w, r1_b, r2_w, r2_b, r3_w, r3_b):
    n = x.shape[0]
    e = edge_index.shape[1]
    src = edge_index[0]
    dst = edge_index[1]

    rkey = dst * n + src
    ar = jnp.arange(e, dtype=jnp.int32)
    rkeyv, p = lax.sort_key_val(rkey, ar)
    dstp = rkeyv // n
    srcp = rkeyv - dstp * n
    keyp = srcp * n + dstp
    jj = jnp.clip(jnp.searchsorted(rkeyv, keyp, method='sort'), 0,
                  e - 1).astype(jnp.int32)
    foundp = rkeyv[jj] == keyp
    idx2 = jnp.where(foundp, jj, ar)
    maskp = jnp.pad(foundp.astype(jnp.float32), (0, EP - e)).reshape(EP, 1)
    ilast = jnp.sum(jnp.where(p == e - 1, ar, 0))

    dstp_pad = jnp.pad(dstp, (0, EP - e))
    bounds = jnp.searchsorted(
        dstp, jnp.arange(33, dtype=jnp.int32) * NODES_PER_W).astype(jnp.int32)
    bounds_pad = jnp.pad(
        jnp.stack([bounds[:32], bounds[1:33]], axis=1), ((0, 0), (0, 14)))
    ztab = jnp.zeros((TROWS, D), jnp.float32)

    xpad = jnp.pad(x, ((0, NP - n), (0, 0)))
    xwb, txb = _node_mm2(xpad, Wi_w[:, :D], Wi_b, Wt_w, Wt_b)

    wbd = jnp.kron(jnp.eye(8, dtype=jnp.float32), Wi_w[:, D:].T)
    ea = _ea_kernel(edge_attr.reshape(e // 8, D), wbd)
    h0p = _sc_gather_add(xwb, srcp, ea, p, EP)

    hh1, hhw = _relu_mm2(h0p, Wh_w)
    agg = _sc_segsum(hh1, dstp_pad, bounds_pad, ztab, 'seg1')
    bnode = _node_mm1(agg, Wh_w, Wh_b)
    g1 = _sc_gather(bnode, srcp, EP, 'g1')
    g2 = _sc_gather(hhw, idx2, EP, 'g2')
    c1r = lax.dynamic_slice_in_dim(hhw, ilast, 1)
    hh2, hhw2 = _ewmm(h0p, g1, g2, maskp, c1r, Wh_w)
    agg2 = _sc_segsum(hh2, dstp_pad, bounds_pad, ztab, 'seg2')
    bnode2 = _node_mm1(agg2, Wh_w, Wh_b)
    g1b = _sc_gather(bnode2, srcp, EP, 'g1b')
    g2b = _sc_gather(hhw2, idx2, EP, 'g2b')
    c2r = lax.dynamic_slice_in_dim(hhw2, ilast, 1)
    hh3 = _ew(h0p, g1b, g2b, maskp, c2r)

    m = _sc_segsum(hh3, dstp_pad, bounds_pad, ztab, 'seg3')

    c3p = jnp.zeros((D, 64), jnp.float32).at[0:2].set(c3_w)
    c3bp = jnp.zeros((D,), jnp.float32).at[0:2].set(c3_b)
    r3p = jnp.zeros((D, 64), jnp.float32).at[2:3].set(r3_w)
    r3bp = jnp.zeros((D,), jnp.float32).at[2:3].set(r3_b)

    out = _head(xpad, m, txb, Wo_w[:, :D], Wo_w[:, D:], Wo_b,
                c1_w, c1_b, c2_w, c2_b, c3p, c3bp,
                r1_w, r1_b, r2_w, r2_b, r3p, r3bp)
    return out[:n, :3]


# speedup vs baseline: 1.0941x; 1.0941x over previous
import functools

import jax
import jax.numpy as jnp
from jax import lax
from jax.experimental import pallas as pl
from jax.experimental.pallas import tpu as pltpu
from jax.experimental.pallas import tpu_sc as plsc

N_NODES = 10000
N_EDGES = 320000
D = 128
NW = 32
NODES_PER_W = 320
NP = NW * NODES_PER_W
EP = N_EDGES + 2560

EDGE_BM = 2560
NODE_BM = 2048

TSTRIDE = NODES_PER_W + 8
TROWS = 16 * TSTRIDE


_DN = (((1,), (1,)), ((), ()))


def _mm2_body(x_ref, wa_ref, ba_ref, wb_ref, bb_ref, oa_ref, ob_ref):
    xv = x_ref[...]
    oa_ref[...] = lax.dot_general(xv, wa_ref[...], _DN,
                                  preferred_element_type=jnp.float32) + ba_ref[...]
    ob_ref[...] = lax.dot_general(xv, wb_ref[...], _DN,
                                  preferred_element_type=jnp.float32) + bb_ref[...]


def _node_mm2(x, wa, ba, wb, bb):
    grid = (x.shape[0] // NODE_BM,)
    return pl.pallas_call(
        _mm2_body,
        grid=grid,
        in_specs=[
            pl.BlockSpec((NODE_BM, D), lambda i: (i, 0)),
            pl.BlockSpec((D, D), lambda i: (0, 0)),
            pl.BlockSpec((1, D), lambda i: (0, 0)),
            pl.BlockSpec((D, D), lambda i: (0, 0)),
            pl.BlockSpec((1, D), lambda i: (0, 0)),
        ],
        out_specs=[
            pl.BlockSpec((NODE_BM, D), lambda i: (i, 0)),
            pl.BlockSpec((NODE_BM, D), lambda i: (i, 0)),
        ],
        out_shape=[
            jax.ShapeDtypeStruct((x.shape[0], D), jnp.float32),
            jax.ShapeDtypeStruct((x.shape[0], D), jnp.float32),
        ],
    )(x, wa, ba.reshape(1, D), wb, bb.reshape(1, D))


def _mm1_body(x_ref, w_ref, b_ref, o_ref):
    o_ref[...] = lax.dot_general(x_ref[...], w_ref[...], _DN,
                                 preferred_element_type=jnp.float32) + b_ref[...]


def _node_mm1(x, w, b):
    grid = (x.shape[0] // NODE_BM,)
    return pl.pallas_call(
        _mm1_body,
        grid=grid,
        in_specs=[
            pl.BlockSpec((NODE_BM, D), lambda i: (i, 0)),
            pl.BlockSpec((D, D), lambda i: (0, 0)),
            pl.BlockSpec((1, D), lambda i: (0, 0)),
        ],
        out_specs=pl.BlockSpec((NODE_BM, D), lambda i: (i, 0)),
        out_shape=jax.ShapeDtypeStruct((x.shape[0], D), jnp.float32),
    )(x, w, b.reshape(1, D))


def _ea_body(ea8_ref, wbd_ref, o_ref):
    t = lax.dot_general(ea8_ref[...], wbd_ref[...], (((1,), (0,)), ((), ())),
                        preferred_element_type=jnp.float32)
    o_ref[...] = t.reshape(EDGE_BM, D)


def _ea_kernel(ea8, wbd):
    grid = (N_EDGES // EDGE_BM,)
    return pl.pallas_call(
        _ea_body,
        grid=grid,
        in_specs=[
            pl.BlockSpec((EDGE_BM // 8, D), lambda i: (i, 0)),
            pl.BlockSpec((D, 8 * D), lambda i: (0, 0)),
        ],
        out_specs=pl.BlockSpec((EDGE_BM, D), lambda i: (i, 0)),
        out_shape=jax.ShapeDtypeStruct((N_EDGES, D), jnp.float32),
    )(ea8, wbd)


def _relu_mm2_body(h0_ref, w_ref, oh_ref, omm_ref):
    t = jnp.maximum(h0_ref[...], 0.0)
    oh_ref[...] = t
    omm_ref[...] = lax.dot_general(t, w_ref[...], _DN,
                                   preferred_element_type=jnp.float32)


def _relu_mm2(h0, w):
    grid = (EP // EDGE_BM,)
    return pl.pallas_call(
        _relu_mm2_body,
        grid=grid,
        in_specs=[
            pl.BlockSpec((EDGE_BM, D), lambda i: (i, 0)),
            pl.BlockSpec((D, D), lambda i: (0, 0)),
        ],
        out_specs=[
            pl.BlockSpec((EDGE_BM, D), lambda i: (i, 0)),
            pl.BlockSpec((EDGE_BM, D), lambda i: (i, 0)),
        ],
        out_shape=[
            jax.ShapeDtypeStruct((EP, D), jnp.float32),
            jax.ShapeDtypeStruct((EP, D), jnp.float32),
        ],
    )(h0, w)


def _ewmm_body(h0_ref, g1_ref, g2_ref, m_ref, c_ref, w_ref, oh_ref, omm_ref):
    g2e = c_ref[...] + m_ref[...] * (g2_ref[...] - c_ref[...])
    t = jnp.maximum(h0_ref[...] + g1_ref[...] - g2e, 0.0)
    oh_ref[...] = t
    omm_ref[...] = lax.dot_general(t, w_ref[...], _DN,
                                   preferred_element_type=jnp.float32)


def _ewmm(h0, g1, g2, m, c, w):
    grid = (EP // EDGE_BM,)
    return pl.pallas_call(
        _ewmm_body,
        grid=grid,
        in_specs=[
            pl.BlockSpec((EDGE_BM, D), lambda i: (i, 0)),
            pl.BlockSpec((EDGE_BM, D), lambda i: (i, 0)),
            pl.BlockSpec((EDGE_BM, D), lambda i: (i, 0)),
            pl.BlockSpec((EDGE_BM, 1), lambda i: (i, 0)),
            pl.BlockSpec((1, D), lambda i: (0, 0)),
            pl.BlockSpec((D, D), lambda i: (0, 0)),
        ],
        out_specs=[
            pl.BlockSpec((EDGE_BM, D), lambda i: (i, 0)),
            pl.BlockSpec((EDGE_BM, D), lambda i: (i, 0)),
        ],
        out_shape=[
            jax.ShapeDtypeStruct((EP, D), jnp.float32),
            jax.ShapeDtypeStruct((EP, D), jnp.float32),
        ],
    )(h0, g1, g2, m, c, w)


def _ew_body(h0_ref, g1_ref, g2_ref, m_ref, c_ref, o_ref):
    g2e = c_ref[...] + m_ref[...] * (g2_ref[...] - c_ref[...])
    o_ref[...] = jnp.maximum(h0_ref[...] + g1_ref[...] - g2e, 0.0)


def _ew(h0, g1, g2, m, c):
    grid = (EP // EDGE_BM,)
    return pl.pallas_call(
        _ew_body,
        grid=grid,
        in_specs=[
            pl.BlockSpec((EDGE_BM, D), lambda i: (i, 0)),
            pl.BlockSpec((EDGE_BM, D), lambda i: (i, 0)),
            pl.BlockSpec((EDGE_BM, D), lambda i: (i, 0)),
            pl.BlockSpec((EDGE_BM, 1), lambda i: (i, 0)),
            pl.BlockSpec((1, D), lambda i: (0, 0)),
        ],
        out_specs=pl.BlockSpec((EDGE_BM, D), lambda i: (i, 0)),
        out_shape=jax.ShapeDtypeStruct((EP, D), jnp.float32),
    )(h0, g1, g2, m, c)


def _head_body(x_ref, m_ref, tx_ref, wox_ref, wom_ref, bo_ref,
               c1_ref, c1b_ref, c2_ref, c2b_ref, c3_ref, c3b_ref,
               r1_ref, r1b_ref, r2_ref, r2b_ref, r3_ref, r3b_ref, o_ref):
    m = m_ref[...]
    msum = jnp.sum(m, axis=1, keepdims=True)
    muse = jnp.where(msum == 0.0, tx_ref[...], m)
    emb = jnp.maximum(
        lax.dot_general(x_ref[...], wox_ref[...], _DN, preferred_element_type=jnp.float32)
        + lax.dot_general(muse, wom_ref[...], _DN, preferred_element_type=jnp.float32)
        + bo_ref[...], 0.0)
    h1 = jnp.maximum(lax.dot_general(emb, c1_ref[...], _DN,
                                     preferred_element_type=jnp.float32) + c1b_ref[...], 0.0)
    h2 = jnp.maximum(lax.dot_general(h1, c2_ref[...], _DN,
                                     preferred_element_type=jnp.float32) + c2b_ref[...], 0.0)
    cls = lax.dot_general(h2, c3_ref[...], _DN,
                          preferred_element_type=jnp.float32) + c3b_ref[...]
    q1 = jnp.maximum(lax.dot_general(emb, r1_ref[...], _DN,
                                     preferred_element_type=jnp.float32) + r1b_ref[...], 0.0)
    q2 = jnp.maximum(lax.dot_general(q1, r2_ref[...], _DN,
                                     preferred_element_type=jnp.float32) + r2b_ref[...], 0.0)
    reg = lax.dot_general(q2, r3_ref[...], _DN,
                          preferred_element_type=jnp.float32) + r3b_ref[...]
    reg = jnp.clip(jax.nn.sigmoid(reg), 0.0, 14.0)
    col = lax.broadcasted_iota(jnp.int32, cls.shape, 1)
    o_ref[...] = jnp.where(col < 2, cls, jnp.where(col == 2, reg, 0.0))


def _head(x, m, tx, wox, wom, bo, c1, c1b, c2, c2b, c3p, c3bp,
          r1, r1b, r2, r2b, r3p, r3bp):
    grid = (NP // NODE_BM,)
    bs = lambda shp: pl.BlockSpec(shp, lambda i: (0, 0))
    return pl.pallas_call(
        _head_body,
        grid=grid,
        in_specs=[
            pl.BlockSpec((NODE_BM, D), lambda i: (i, 0)),
            pl.BlockSpec((NODE_BM, D), lambda i: (i, 0)),
            pl.BlockSpec((NODE_BM, D), lambda i: (i, 0)),
            bs((D, D)), bs((D, D)), bs((1, D)),
            bs((256, D)), bs((1, 256)), bs((64, 256)), bs((1, 64)),
            bs((D, 64)), bs((1, D)),
            bs((256, D)), bs((1, 256)), bs((64, 256)), bs((1, 64)),
            bs((D, 64)), bs((1, D)),
        ],
        out_specs=pl.BlockSpec((NODE_BM, D), lambda i: (i, 0)),
        out_shape=jax.ShapeDtypeStruct((NP, D), jnp.float32),
    )(x, m, tx, wox, wom, bo.reshape(1, D),
      c1, c1b.reshape(1, 256), c2, c2b.reshape(1, 64), c3p, c3bp.reshape(1, D),
      r1, r1b.reshape(1, 256), r2, r2b.reshape(1, 64), r3p, r3bp.reshape(1, D))


_MESH = functools.partial(plsc.VectorSubcoreMesh,
                          core_axis_name="c", subcore_axis_name="s")

_GCH = 400
_GSUB = ((0, 128), (128, 128), (256, 128), (384, 16))


def _sc_gather(table, idx, out_rows, name):
    d = table.shape[1]
    e = idx.shape[0]
    per_w = e // NW
    n_iter = per_w // _GCH

    def body(table_ref, idx_ref, out_ref, i0, i1, i2, i3, rowv, sem):
        c = lax.axis_index("c")
        s = lax.axis_index("s")
        wid = s * 2 + c
        base0 = wid * per_w
        ibufs = (i0, i1, i2, i3)

        def it(j, carry):
            off = base0 + j * _GCH
            for (k, ln), ib in zip(_GSUB, ibufs):
                pltpu.sync_copy(idx_ref.at[pl.ds(off + k, ln)], ib)
            cps = [pltpu.async_copy(table_ref.at[ib], rowv.at[pl.ds(k, ln)], sem)
                   for (k, ln), ib in zip(_GSUB, ibufs)]
            for cp in cps:
                cp.wait()
            pltpu.sync_copy(rowv, out_ref.at[pl.ds(off, _GCH)])
            return carry

        lax.fori_loop(0, n_iter, it, 0)

    return pl.kernel(
        body,
        out_type=jax.ShapeDtypeStruct((out_rows, d), jnp.float32),
        mesh=_MESH(),
        name=name,
        scratch_types=[
            pltpu.VMEM((128,), jnp.int32),
            pltpu.VMEM((128,), jnp.int32),
            pltpu.VMEM((128,), jnp.int32),
            pltpu.VMEM((16,), jnp.int32),
            pltpu.VMEM((_GCH, d), jnp.float32),
            pltpu.SemaphoreType.DMA,
        ],
    )(table, idx)


def _sc_gather_add(ta, ia, tb, ib_idx, out_rows):
    name = 'scgadd'
    d = ta.shape[1]
    e = ia.shape[0]
    per_w = e // NW
    n_iter = per_w // _GCH

    def body(ta_ref, ia_ref, tb_ref, ib_ref, out_ref, i0, i1, i2, i3, rowv, sem):
        c = lax.axis_index("c")
        s = lax.axis_index("s")
        wid = s * 2 + c
        base0 = wid * per_w
        ibufs = (i0, i1, i2, i3)

        def it(j, carry):
            off = base0 + j * _GCH
            for (k, ln), ib in zip(_GSUB, ibufs):
                pltpu.sync_copy(ia_ref.at[pl.ds(off + k, ln)], ib)
            cps = [pltpu.async_copy(ta_ref.at[ib], rowv.at[pl.ds(k, ln)], sem)
                   for (k, ln), ib in zip(_GSUB, ibufs)]
            for cp in cps:
                cp.wait()
            for (k, ln), ib in zip(_GSUB, ibufs):
                pltpu.sync_copy(ib_ref.at[pl.ds(off + k, ln)], ib)
            cps = [pltpu.async_copy(tb_ref.at[ib], rowv.at[pl.ds(k, ln)], sem,
                                    add=True)
                   for (k, ln), ib in zip(_GSUB, ibufs)]
            for cp in cps:
                cp.wait()
            pltpu.sync_copy(rowv, out_ref.at[pl.ds(off, _GCH)])
            return carry

        lax.fori_loop(0, n_iter, it, 0)

    return pl.kernel(
        body,
        out_type=jax.ShapeDtypeStruct((out_rows, d), jnp.float32),
        mesh=_MESH(),
        name=name,
        scratch_types=[
            pltpu.VMEM((128,), jnp.int32),
            pltpu.VMEM((128,), jnp.int32),
            pltpu.VMEM((128,), jnp.int32),
            pltpu.VMEM((16,), jnp.int32),
            pltpu.VMEM((_GCH, d), jnp.float32),
            pltpu.SemaphoreType.DMA,
        ],
    )(ta, ia, tb, ib_idx)


def _ext48(v0, v1, v2, i):
    z = jnp.int32(0)
    io = lax.iota(jnp.int32, 16)
    r0 = jnp.sum(jnp.where(io == i, v0, z))
    r1 = jnp.sum(jnp.where(io == (i - 16), v1, z))
    r2 = jnp.sum(jnp.where(io == (i - 32), v2, z))
    return r0 + r1 + r2


def _sc_segsum(rows, dstp_pad, bounds_pad, ztab, name):
    def body(rows_ref, dst_ref, bnd_ref, z_ref, out_ref,
             bndv, dstv, idxv, rowv, table):
        c = lax.axis_index("c")
        s = lax.axis_index("s")
        wid = s * 2 + c
        tb = TSTRIDE * s
        dump = tb + NODES_PER_W

        pltpu.sync_copy(z_ref.at[pl.ds(tb, TSTRIDE)], table.at[pl.ds(tb, TSTRIDE)])
        pltpu.sync_copy(bnd_ref.at[wid], bndv)
        bv = bndv[...]
        lo = bv[0]
        hi = bv[1]
        a = (lo // 128) * 128
        nch = (hi - a + 127) // 128
        shift = tb - NODES_PER_W * wid

        def it(j, carry):
            base = a + j * 128
            pltpu.sync_copy(rows_ref.at[pl.ds(base, 128), :], rowv)
            pltpu.sync_copy(dst_ref.at[pl.ds(base, 128)], dstv)
            for g in range(8):
                ids = base + 16 * g + lax.iota(jnp.int32, 16)
                dv = dstv[pl.ds(16 * g, 16)]
                ok = (ids >= lo) & (ids < hi)
                idxv[pl.ds(16 * g, 16)] = jnp.where(ok, dv + shift, dump)
            pltpu.sync_copy(rowv, table.at[idxv], add=True)
            return carry

        lax.fori_loop(0, nch, it, 0)
        pltpu.sync_copy(table.at[pl.ds(tb, NODES_PER_W)],
                        out_ref.at[pl.ds(wid * NODES_PER_W, NODES_PER_W)])

    return pl.kernel(
        body,
        out_type=jax.ShapeDtypeStruct((NP, D), jnp.float32),
        mesh=_MESH(),
        name=name,
        scratch_types=[
            pltpu.VMEM((16,), jnp.int32),
            pltpu.VMEM((128,), jnp.int32),
            pltpu.VMEM((128,), jnp.int32),
            pltpu.VMEM((128, D), jnp.float32),
            pltpu.VMEM_SHARED((TROWS, D), jnp.float32),
        ],
    )(rows, dstp_pad, bounds_pad, ztab)


def kernel(x, edge_index, edge_attr, dissociable_masks, pka_values,
           Wi_w, Wi_b, Wh_w, Wh_b, Wo_w, Wo_b, Wt_w, Wt_b,
           c1_w, c1_b, c2_w, c2_b, c3_w, c3_b,
           r1_w, r1_b, r2_w, r2_b, r3_w, r3_b):
    n = x.shape[0]
    e = edge_index.shape[1]
    src = edge_index[0]
    dst = edge_index[1]

    rkey = dst * n + src
    ar = jnp.arange(e, dtype=jnp.int32)
    rkeyv, p = lax.sort_key_val(rkey, ar)
    dstp = rkeyv // n
    srcp = rkeyv - dstp * n
    keyp = srcp * n + dstp
    jj = jnp.clip(jnp.searchsorted(rkeyv, keyp), 0, e - 1).astype(jnp.int32)
    foundp = rkeyv[jj] == keyp
    idx2 = jnp.where(foundp, jj, ar)
    maskp = jnp.pad(foundp.astype(jnp.float32), (0, EP - e)).reshape(EP, 1)
    ilast = jnp.sum(jnp.where(p == e - 1, ar, 0))

    dstp_pad = jnp.pad(dstp, (0, EP - e))
    bounds = jnp.searchsorted(
        dstp, jnp.arange(33, dtype=jnp.int32) * NODES_PER_W).astype(jnp.int32)
    bounds_pad = jnp.pad(
        jnp.stack([bounds[:32], bounds[1:33]], axis=1), ((0, 0), (0, 14)))
    ztab = jnp.zeros((TROWS, D), jnp.float32)

    xpad = jnp.pad(x, ((0, NP - n), (0, 0)))
    xwb, txb = _node_mm2(xpad, Wi_w[:, :D], Wi_b, Wt_w, Wt_b)

    wbd = jnp.kron(jnp.eye(8, dtype=jnp.float32), Wi_w[:, D:].T)
    ea = _ea_kernel(edge_attr.reshape(e // 8, D), wbd)
    h0p = _sc_gather_add(xwb, srcp, ea, p, EP)

    hh1, hhw = _relu_mm2(h0p, Wh_w)
    agg = _sc_segsum(hh1, dstp_pad, bounds_pad, ztab, 'seg1')
    bnode = _node_mm1(agg, Wh_w, Wh_b)
    g1 = _sc_gather(bnode, srcp, EP, 'g1')
    g2 = _sc_gather(hhw, idx2, EP, 'g2')
    c1r = lax.dynamic_slice_in_dim(hhw, ilast, 1)
    hh2, hhw2 = _ewmm(h0p, g1, g2, maskp, c1r, Wh_w)
    agg2 = _sc_segsum(hh2, dstp_pad, bounds_pad, ztab, 'seg2')
    bnode2 = _node_mm1(agg2, Wh_w, Wh_b)
    g1b = _sc_gather(bnode2, srcp, EP, 'g1b')
    g2b = _sc_gather(hhw2, idx2, EP, 'g2b')
    c2r = lax.dynamic_slice_in_dim(hhw2, ilast, 1)
    hh3 = _ew(h0p, g1b, g2b, maskp, c2r)

    m = _sc_segsum(hh3, dstp_pad, bounds_pad, ztab, 'seg3')

    c3p = jnp.zeros((D, 64), jnp.float32).at[0:2].set(c3_w)
    c3bp = jnp.zeros((D,), jnp.float32).at[0:2].set(c3_b)
    r3p = jnp.zeros((D, 64), jnp.float32).at[2:3].set(r3_w)
    r3bp = jnp.zeros((D,), jnp.float32).at[2:3].set(r3_b)

    out = _head(xpad, m, txb, Wo_w[:, :D], Wo_w[:, D:], Wo_b,
                c1_w, c1_b, c2_w, c2_b, c3p, c3bp,
                r1_w, r1_b, r2_w, r2_b, r3p, r3bp)
    return out[:n, :3]
